# Initial kernel scaffold; baseline (speedup 1.0000x reference)
import functools

import jax
import jax.numpy as jnp
from jax import lax
from jax.experimental import pallas as pl
from jax.experimental.pallas import tpu as pltpu
from jax.experimental.pallas import tpu_sc as plsc

N_NODES = 10000
D_FEAT = 128
OUT_CH = 128
N_EDGES = 320000

NC = 2
NS = 16
NW = NC * NS
E_PER_TILE = N_EDGES // NW
CHUNK = 125
NCHUNK = E_PER_TILE // CHUNK
ROWS_PER_TILE = N_NODES // NS
CW = 16

_MESH = plsc.VectorSubcoreMesh(
    core_axis_name="c", subcore_axis_name="s", num_cores=NC, num_subcores=NS
)


def _zero_rows(ref, nrows, ncols):
    zv = jnp.zeros((16,), jnp.float32)

    def body(i, _):
        for k in range(ncols // 16):
            ref[i, pl.ds(k * 16, 16)] = zv
        return 0

    lax.fori_loop(0, nrows, body, 0)


@functools.partial(
    pl.kernel,
    out_type=jax.ShapeDtypeStruct((NC, N_NODES, CW), jnp.float32),
    mesh=_MESH,
    scratch_types=[
        pltpu.VMEM((NCHUNK, CHUNK), jnp.int32),
        pltpu.VMEM((CHUNK, CW), jnp.float32),
        pltpu.VMEM((CHUNK, CW), jnp.float32),
        pltpu.VMEM_SHARED((N_NODES, CW), jnp.float32),
    ],
)
def _sc_degree(dst_hbm, out_hbm, dst_v, ones_v, zeros_v, cnt_sh):
    cid = lax.axis_index("c")
    sid = lax.axis_index("s")
    wid = cid * NS + sid

    pltpu.sync_copy(dst_hbm.at[wid], dst_v)

    ov = jnp.ones((16,), jnp.float32)

    def fill(i, _):
        ones_v[i] = ov
        zeros_v[i] = ov * 0.0
        return 0

    lax.fori_loop(0, CHUNK, fill, 0)

    for k in range(ROWS_PER_TILE // CHUNK):
        pltpu.sync_copy(zeros_v, cnt_sh.at[pl.ds(sid * ROWS_PER_TILE + k * CHUNK, CHUNK)])
    plsc.subcore_barrier()

    def step(j, _):
        pltpu.sync_copy(ones_v, cnt_sh.at[dst_v.at[j]], add=True)
        return 0

    lax.fori_loop(0, NCHUNK, step, 0)
    plsc.subcore_barrier()

    pltpu.sync_copy(
        cnt_sh.at[pl.ds(sid * ROWS_PER_TILE, ROWS_PER_TILE)],
        out_hbm.at[cid, pl.ds(sid * ROWS_PER_TILE, ROWS_PER_TILE)],
    )


@functools.partial(
    pl.kernel,
    out_type=jax.ShapeDtypeStruct((NC, N_NODES, OUT_CH), jnp.float32),
    mesh=_MESH,
    scratch_types=[
        pltpu.VMEM((NCHUNK, CHUNK), jnp.int32),
        pltpu.VMEM((NCHUNK, CHUNK), jnp.int32),
        pltpu.VMEM((CHUNK, OUT_CH), jnp.float32),
        pltpu.VMEM_SHARED((N_NODES, OUT_CH), jnp.float32),
        pltpu.SemaphoreType.DMA,
    ],
)
def _sc_aggregate(src_hbm, dst_hbm, hp_hbm, out_hbm, src_v, dst_v, rows_v, acc_sh, sem):
    cid = lax.axis_index("c")
    sid = lax.axis_index("s")
    wid = cid * NS + sid

    pltpu.sync_copy(src_hbm.at[wid], src_v)
    pltpu.sync_copy(dst_hbm.at[wid], dst_v)

    _zero_rows(rows_v, CHUNK, OUT_CH)
    for k in range(ROWS_PER_TILE // CHUNK):
        pltpu.sync_copy(rows_v, acc_sh.at[pl.ds(sid * ROWS_PER_TILE + k * CHUNK, CHUNK)])
    plsc.subcore_barrier()

    def step(j, _):
        pltpu.async_copy(hp_hbm.at[src_v.at[j]], rows_v, sem).wait()
        pltpu.sync_copy(rows_v, acc_sh.at[dst_v.at[j]], add=True)
        return 0

    lax.fori_loop(0, NCHUNK, step, 0)
    plsc.subcore_barrier()

    pltpu.sync_copy(
        acc_sh.at[pl.ds(sid * ROWS_PER_TILE, ROWS_PER_TILE)],
        out_hbm.at[cid, pl.ds(sid * ROWS_PER_TILE, ROWS_PER_TILE)],
    )


BLK = 200
GRID = N_NODES // BLK


def _tc_matmul_body(x_ref, w_ref, degp_ref, hp_ref):
    deg = degp_ref[0, :, 0] + degp_ref[1, :, 0] + 1.0
    dis = lax.rsqrt(deg)
    h = jnp.dot(x_ref[...], w_ref[...], preferred_element_type=jnp.float32)
    hp_ref[...] = h * dis[:, None]


def _tc_epilogue_body(acc_ref, hp_ref, degp_ref, b_ref, o_ref):
    deg = degp_ref[0, :, 0] + degp_ref[1, :, 0] + 1.0
    dis = lax.rsqrt(deg)
    s = (acc_ref[0] + acc_ref[1] + hp_ref[...]) * dis[:, None] + b_ref[...]
    o_ref[...] = jnp.maximum(s, 0.0)


def kernel(x, edge_index, W, b):
    ei = edge_index.astype(jnp.int32)
    src3 = ei[0].reshape(NW, NCHUNK, CHUNK)
    dst3 = ei[1].reshape(NW, NCHUNK, CHUNK)

    degp = _sc_degree(dst3)

    hp = pl.pallas_call(
        _tc_matmul_body,
        grid=(GRID,),
        in_specs=[
            pl.BlockSpec((BLK, D_FEAT), lambda i: (i, 0)),
            pl.BlockSpec((D_FEAT, OUT_CH), lambda i: (0, 0)),
            pl.BlockSpec((NC, BLK, CW), lambda i: (0, i, 0)),
        ],
        out_specs=pl.BlockSpec((BLK, OUT_CH), lambda i: (i, 0)),
        out_shape=jax.ShapeDtypeStruct((N_NODES, OUT_CH), jnp.float32),
    )(x, W, degp)

    acc = _sc_aggregate(src3, dst3, hp)

    out = pl.pallas_call(
        _tc_epilogue_body,
        grid=(GRID,),
        in_specs=[
            pl.BlockSpec((NC, BLK, OUT_CH), lambda i: (0, i, 0)),
            pl.BlockSpec((BLK, OUT_CH), lambda i: (i, 0)),
            pl.BlockSpec((NC, BLK, CW), lambda i: (0, i, 0)),
            pl.BlockSpec((1, OUT_CH), lambda i: (0, 0)),
        ],
        out_specs=pl.BlockSpec((BLK, OUT_CH), lambda i: (i, 0)),
        out_shape=jax.ShapeDtypeStruct((N_NODES, OUT_CH), jnp.float32),
    )(acc, hp, degp, b.reshape(1, OUT_CH))

    return out


# speedup vs baseline: 12.2828x; 12.2828x over previous
import functools

import jax
import jax.numpy as jnp
from jax import lax
from jax.experimental import pallas as pl
from jax.experimental.pallas import tpu as pltpu
from jax.experimental.pallas import tpu_sc as plsc

N_NODES = 10000
D_FEAT = 128
OUT_CH = 128
N_EDGES = 320000

NC = 2
NS = 16
NW = NC * NS
CHUNK = 128
NCHUNK = 80
E_PER_TILE = CHUNK * NCHUNK
E_PAD = E_PER_TILE * NW
N_PAD = 10240
ROWS_PT = N_PAD // NS
CW = 16

_MESH = plsc.VectorSubcoreMesh(
    core_axis_name="c", subcore_axis_name="s", num_cores=NC, num_subcores=NS
)


def _fill_rows(ref, nrows, ncols, value):
    vv = jnp.full((16,), value, jnp.float32)

    def body(i, _):
        for k in range(ncols // 16):
            ref[i, pl.ds(k * 16, 16)] = vv
        return 0

    lax.fori_loop(0, nrows, body, 0)


@functools.partial(
    pl.kernel,
    out_type=jax.ShapeDtypeStruct((NC, N_PAD, CW), jnp.float32),
    mesh=_MESH,
    scratch_types=[
        pltpu.VMEM((NCHUNK, CHUNK), jnp.int32),
        pltpu.VMEM((CHUNK, CW), jnp.float32),
        pltpu.VMEM((CHUNK, CW), jnp.float32),
        pltpu.VMEM_SHARED((N_PAD, CW), jnp.float32),
    ],
)
def _sc_degree(dst_hbm, out_hbm, dst_v, ones_v, zeros_v, cnt_sh):
    cid = lax.axis_index("c")
    sid = lax.axis_index("s")
    wid = cid * NS + sid

    pltpu.sync_copy(dst_hbm.at[wid], dst_v)
    _fill_rows(ones_v, CHUNK, CW, 1.0)
    _fill_rows(zeros_v, CHUNK, CW, 0.0)

    for k in range(ROWS_PT // CHUNK):
        pltpu.sync_copy(zeros_v, cnt_sh.at[pl.ds(sid * ROWS_PT + k * CHUNK, CHUNK)])
    plsc.subcore_barrier()

    def step(j, _):
        pltpu.sync_copy(ones_v, cnt_sh.at[dst_v.at[j]], add=True)
        return 0

    lax.fori_loop(0, NCHUNK, step, 0)
    plsc.subcore_barrier()

    pltpu.sync_copy(
        cnt_sh.at[pl.ds(sid * ROWS_PT, ROWS_PT)],
        out_hbm.at[cid, pl.ds(sid * ROWS_PT, ROWS_PT)],
    )


@functools.partial(
    pl.kernel,
    out_type=jax.ShapeDtypeStruct((NC, N_PAD, OUT_CH), jnp.float32),
    mesh=_MESH,
    scratch_types=[
        pltpu.VMEM((NCHUNK, CHUNK), jnp.int32),
        pltpu.VMEM((NCHUNK, CHUNK), jnp.int32),
        pltpu.VMEM((CHUNK, OUT_CH), jnp.float32),
        pltpu.VMEM_SHARED((N_PAD, OUT_CH), jnp.float32),
        pltpu.SemaphoreType.DMA,
    ],
)
def _sc_aggregate(src_hbm, dst_hbm, hp_hbm, out_hbm,
                  src_v, dst_v, rows_v, acc_sh, sem):
    cid = lax.axis_index("c")
    sid = lax.axis_index("s")
    wid = cid * NS + sid

    pltpu.sync_copy(src_hbm.at[wid], src_v)
    pltpu.sync_copy(dst_hbm.at[wid], dst_v)

    _fill_rows(rows_v, CHUNK, OUT_CH, 0.0)
    for k in range(ROWS_PT // CHUNK):
        pltpu.sync_copy(rows_v, acc_sh.at[pl.ds(sid * ROWS_PT + k * CHUNK, CHUNK)])
    plsc.subcore_barrier()

    def step(j, _):
        pltpu.async_copy(hp_hbm.at[src_v.at[j]], rows_v, sem).wait()
        pltpu.sync_copy(rows_v, acc_sh.at[dst_v.at[j]], add=True)
        return 0

    lax.fori_loop(0, NCHUNK, step, 0)
    plsc.subcore_barrier()

    pltpu.sync_copy(
        acc_sh.at[pl.ds(sid * ROWS_PT, ROWS_PT)],
        out_hbm.at[cid, pl.ds(sid * ROWS_PT, ROWS_PT)],
    )


BLK = 200
GRID = N_NODES // BLK


def _tc_matmul_body(x_ref, w_ref, degp_ref, hp_ref):
    deg = degp_ref[0, :, 0] + degp_ref[1, :, 0] + 1.0
    dis = lax.rsqrt(deg)
    h = jnp.dot(x_ref[...], w_ref[...], preferred_element_type=jnp.float32)
    hp_ref[...] = h * dis[:, None]


def _tc_epilogue_body(acc_ref, hp_ref, degp_ref, b_ref, o_ref):
    deg = degp_ref[0, :, 0] + degp_ref[1, :, 0] + 1.0
    dis = lax.rsqrt(deg)
    s = (acc_ref[0] + acc_ref[1] + hp_ref[...]) * dis[:, None] + b_ref[...]
    o_ref[...] = jnp.maximum(s, 0.0)


def kernel(x, edge_index, W, b):
    ei = edge_index.astype(jnp.int32)
    npad = E_PAD - N_EDGES
    src3 = jnp.concatenate(
        [ei[0], jnp.zeros((npad,), jnp.int32)]).reshape(NW, NCHUNK, CHUNK)
    dst3 = jnp.concatenate(
        [ei[1], jnp.full((npad,), N_NODES, jnp.int32)]).reshape(NW, NCHUNK, CHUNK)

    degp = _sc_degree(dst3)

    hp = pl.pallas_call(
        _tc_matmul_body,
        grid=(GRID,),
        in_specs=[
            pl.BlockSpec((BLK, D_FEAT), lambda i: (i, 0)),
            pl.BlockSpec((D_FEAT, OUT_CH), lambda i: (0, 0)),
            pl.BlockSpec((NC, BLK, CW), lambda i: (0, i, 0)),
        ],
        out_specs=pl.BlockSpec((BLK, OUT_CH), lambda i: (i, 0)),
        out_shape=jax.ShapeDtypeStruct((N_NODES, OUT_CH), jnp.float32),
    )(x, W, degp)

    acc = _sc_aggregate(src3, dst3, hp)

    out = pl.pallas_call(
        _tc_epilogue_body,
        grid=(GRID,),
        in_specs=[
            pl.BlockSpec((NC, BLK, OUT_CH), lambda i: (0, i, 0)),
            pl.BlockSpec((BLK, OUT_CH), lambda i: (i, 0)),
            pl.BlockSpec((NC, BLK, CW), lambda i: (0, i, 0)),
            pl.BlockSpec((1, OUT_CH), lambda i: (0, 0)),
        ],
        out_specs=pl.BlockSpec((BLK, OUT_CH), lambda i: (i, 0)),
        out_shape=jax.ShapeDtypeStruct((N_NODES, OUT_CH), jnp.float32),
    )(acc, hp, degp, b.reshape(1, OUT_CH))

    return out


# speedup vs baseline: 14.1038x; 1.1483x over previous
import functools

import jax
import jax.numpy as jnp
from jax import lax
from jax.experimental import pallas as pl
from jax.experimental.pallas import tpu as pltpu
from jax.experimental.pallas import tpu_sc as plsc

N_NODES = 10000
D_FEAT = 128
OUT_CH = 128
N_EDGES = 320000

NC = 2
NS = 16
NW = NC * NS
CHUNK = 128
NCHUNK = 80
ACH = 64
NACH = 160
E_PER_TILE = CHUNK * NCHUNK
E_PAD = E_PER_TILE * NW
N_PAD = 10240
ROWS_PT = N_PAD // NS
CW = 16

_MESH = plsc.VectorSubcoreMesh(
    core_axis_name="c", subcore_axis_name="s", num_cores=NC, num_subcores=NS
)


def _fill_rows(ref, nrows, ncols, value):
    vv = jnp.full((16,), value, jnp.float32)

    def body(i, _):
        for k in range(ncols // 16):
            ref[i, pl.ds(k * 16, 16)] = vv
        return 0

    lax.fori_loop(0, nrows, body, 0)


@functools.partial(
    pl.kernel,
    out_type=jax.ShapeDtypeStruct((NC, N_PAD, CW), jnp.float32),
    mesh=_MESH,
    scratch_types=[
        pltpu.VMEM((NCHUNK, CHUNK), jnp.int32),
        pltpu.VMEM((CHUNK, CW), jnp.float32),
        pltpu.VMEM((CHUNK, CW), jnp.float32),
        pltpu.VMEM_SHARED((N_PAD, CW), jnp.float32),
    ],
)
def _sc_degree(dst_hbm, out_hbm, dst_v, ones_v, zeros_v, cnt_sh):
    cid = lax.axis_index("c")
    sid = lax.axis_index("s")
    wid = cid * NS + sid

    pltpu.sync_copy(dst_hbm.at[wid], dst_v)
    _fill_rows(ones_v, CHUNK, CW, 1.0)
    _fill_rows(zeros_v, CHUNK, CW, 0.0)

    for k in range(ROWS_PT // CHUNK):
        pltpu.sync_copy(zeros_v, cnt_sh.at[pl.ds(sid * ROWS_PT + k * CHUNK, CHUNK)])
    plsc.subcore_barrier()

    def step(j, _):
        pltpu.sync_copy(ones_v, cnt_sh.at[dst_v.at[j]], add=True)
        return 0

    lax.fori_loop(0, NCHUNK, step, 0)
    plsc.subcore_barrier()

    pltpu.sync_copy(
        cnt_sh.at[pl.ds(sid * ROWS_PT, ROWS_PT)],
        out_hbm.at[cid, pl.ds(sid * ROWS_PT, ROWS_PT)],
    )


@functools.partial(
    pl.kernel,
    out_type=jax.ShapeDtypeStruct((NC, N_PAD, OUT_CH), jnp.float32),
    mesh=_MESH,
    scratch_types=[
        pltpu.VMEM((NCHUNK, CHUNK), jnp.int32),
        pltpu.VMEM((NACH, ACH), jnp.int32),
        pltpu.VMEM((ACH, OUT_CH), jnp.float32),
        pltpu.VMEM((ACH, OUT_CH), jnp.float32),
        pltpu.VMEM_SHARED((N_PAD, OUT_CH), jnp.float32),
        pltpu.SemaphoreType.DMA,
        pltpu.SemaphoreType.DMA,
    ],
)
def _sc_aggregate(src_hbm, dst_hbm, hp_hbm, out_hbm,
                  src_v, dst_v, rows0, rows1, acc_sh, sem0, sem1):
    cid = lax.axis_index("c")
    sid = lax.axis_index("s")
    wid = cid * NS + sid

    pltpu.sync_copy(src_hbm.at[wid], src_v)
    pltpu.sync_copy(dst_hbm.at[wid], dst_v)

    _fill_rows(rows0, ACH, OUT_CH, 0.0)
    for k in range(ROWS_PT // ACH):
        pltpu.sync_copy(rows0, acc_sh.at[pl.ds(sid * ROWS_PT + k * ACH, ACH)])
    plsc.subcore_barrier()

    pltpu.async_copy(hp_hbm.at[src_v.at[0, pl.ds(0, ACH)]], rows0, sem0)

    def step(i, _):
        pltpu.make_async_copy(hp_hbm.at[src_v.at[i, pl.ds(0, ACH)]], rows0, sem0).wait()
        pltpu.async_copy(hp_hbm.at[src_v.at[i, pl.ds(ACH, ACH)]], rows1, sem1)
        pltpu.sync_copy(rows0, acc_sh.at[dst_v.at[2 * i]], add=True)
        pltpu.make_async_copy(hp_hbm.at[src_v.at[i, pl.ds(ACH, ACH)]], rows1, sem1).wait()

        @pl.when(i < NCHUNK - 1)
        def _():
            pltpu.async_copy(hp_hbm.at[src_v.at[i + 1, pl.ds(0, ACH)]], rows0, sem0)

        pltpu.sync_copy(rows1, acc_sh.at[dst_v.at[2 * i + 1]], add=True)
        return 0

    lax.fori_loop(0, NCHUNK, step, 0)
    plsc.subcore_barrier()

    pltpu.sync_copy(
        acc_sh.at[pl.ds(sid * ROWS_PT, ROWS_PT)],
        out_hbm.at[cid, pl.ds(sid * ROWS_PT, ROWS_PT)],
    )


BLK = 200
GRID = N_NODES // BLK


def _tc_matmul_body(x_ref, w_ref, degp_ref, hp_ref):
    deg = degp_ref[0, :, 0] + degp_ref[1, :, 0] + 1.0
    dis = lax.rsqrt(deg)
    h = jnp.dot(x_ref[...], w_ref[...], preferred_element_type=jnp.float32)
    hp_ref[...] = h * dis[:, None]


def _tc_epilogue_body(acc_ref, hp_ref, degp_ref, b_ref, o_ref):
    deg = degp_ref[0, :, 0] + degp_ref[1, :, 0] + 1.0
    dis = lax.rsqrt(deg)
    s = (acc_ref[0] + acc_ref[1] + hp_ref[...]) * dis[:, None] + b_ref[...]
    o_ref[...] = jnp.maximum(s, 0.0)


def kernel(x, edge_index, W, b):
    ei = edge_index.astype(jnp.int32)
    npad = E_PAD - N_EDGES
    src_p = jnp.concatenate([ei[0], jnp.zeros((npad,), jnp.int32)])
    dst_p = jnp.concatenate([ei[1], jnp.full((npad,), N_NODES, jnp.int32)])

    degp = _sc_degree(dst_p.reshape(NW, NCHUNK, CHUNK))
    srcA = src_p.reshape(NW, NCHUNK, CHUNK)
    dstA = dst_p.reshape(NW, NACH, ACH)

    hp = pl.pallas_call(
        _tc_matmul_body,
        grid=(GRID,),
        in_specs=[
            pl.BlockSpec((BLK, D_FEAT), lambda i: (i, 0)),
            pl.BlockSpec((D_FEAT, OUT_CH), lambda i: (0, 0)),
            pl.BlockSpec((NC, BLK, CW), lambda i: (0, i, 0)),
        ],
        out_specs=pl.BlockSpec((BLK, OUT_CH), lambda i: (i, 0)),
        out_shape=jax.ShapeDtypeStruct((N_NODES, OUT_CH), jnp.float32),
    )(x, W, degp)

    acc = _sc_aggregate(srcA, dstA, hp)

    out = pl.pallas_call(
        _tc_epilogue_body,
        grid=(GRID,),
        in_specs=[
            pl.BlockSpec((NC, BLK, OUT_CH), lambda i: (0, i, 0)),
            pl.BlockSpec((BLK, OUT_CH), lambda i: (i, 0)),
            pl.BlockSpec((NC, BLK, CW), lambda i: (0, i, 0)),
            pl.BlockSpec((1, OUT_CH), lambda i: (0, 0)),
        ],
        out_specs=pl.BlockSpec((BLK, OUT_CH), lambda i: (i, 0)),
        out_shape=jax.ShapeDtypeStruct((N_NODES, OUT_CH), jnp.float32),
    )(acc, hp, degp, b.reshape(1, OUT_CH))

    return out


# speedup vs baseline: 16.4548x; 1.1667x over previous
import functools

import jax
import jax.numpy as jnp
from jax import lax
from jax.experimental import pallas as pl
from jax.experimental.pallas import tpu as pltpu
from jax.experimental.pallas import tpu_sc as plsc

N_NODES = 10000
D_FEAT = 128
OUT_CH = 128
N_EDGES = 320000

NC = 2
NS = 16
NW = NC * NS
CHUNK = 128
NCHUNK = 80
ACH = 64
NACH = 160
TOT_CHUNKS = 2560
C0 = 96
C1 = 64
STG = 16
E_PER_TILE = CHUNK * NCHUNK
E_PAD = E_PER_TILE * NW
N_PAD = 10240
ROWS_PT = N_PAD // NS
CW = 16

_MESH = plsc.VectorSubcoreMesh(
    core_axis_name="c", subcore_axis_name="s", num_cores=NC, num_subcores=NS
)


def _fill_rows(ref, nrows, ncols, value):
    vv = jnp.full((16,), value, jnp.float32)

    def body(i, _):
        for k in range(ncols // 16):
            ref[i, pl.ds(k * 16, 16)] = vv
        return 0

    lax.fori_loop(0, nrows, body, 0)


@functools.partial(
    pl.kernel,
    out_type=jax.ShapeDtypeStruct((NC, N_PAD, CW), jnp.float32),
    mesh=_MESH,
    scratch_types=[
        pltpu.VMEM((NCHUNK, CHUNK), jnp.int32),
        pltpu.VMEM((CHUNK, CW), jnp.float32),
        pltpu.VMEM((CHUNK, CW), jnp.float32),
        pltpu.VMEM_SHARED((N_PAD, CW), jnp.float32),
    ],
)
def _sc_degree(dst_hbm, out_hbm, dst_v, ones_v, zeros_v, cnt_sh):
    cid = lax.axis_index("c")
    sid = lax.axis_index("s")
    wid = cid * NS + sid

    pltpu.sync_copy(dst_hbm.at[wid], dst_v)
    _fill_rows(ones_v, CHUNK, CW, 1.0)
    _fill_rows(zeros_v, CHUNK, CW, 0.0)

    for k in range(ROWS_PT // CHUNK):
        pltpu.sync_copy(zeros_v, cnt_sh.at[pl.ds(sid * ROWS_PT + k * CHUNK, CHUNK)])
    plsc.subcore_barrier()

    def step(j, _):
        pltpu.sync_copy(ones_v, cnt_sh.at[dst_v.at[j]], add=True)
        return 0

    lax.fori_loop(0, NCHUNK, step, 0)
    plsc.subcore_barrier()

    pltpu.sync_copy(
        cnt_sh.at[pl.ds(sid * ROWS_PT, ROWS_PT)],
        out_hbm.at[cid, pl.ds(sid * ROWS_PT, ROWS_PT)],
    )


@functools.partial(
    pl.kernel,
    out_type=jax.ShapeDtypeStruct((NC, N_PAD, OUT_CH), jnp.float32),
    mesh=_MESH,
    scratch_types=[
        pltpu.VMEM((STG, CHUNK), jnp.int32),
        pltpu.VMEM((STG, CHUNK), jnp.int32),
        pltpu.VMEM((CHUNK, OUT_CH), jnp.float32),
        pltpu.VMEM((CHUNK, OUT_CH), jnp.float32),
        pltpu.VMEM_SHARED((N_PAD, OUT_CH), jnp.float32),
        pltpu.SemaphoreType.DMA,
        pltpu.SemaphoreType.DMA,
    ],
)
def _sc_aggregate(src_hbm, dst_hbm, hp_hbm, out_hbm,
                  src_v, dst_v, rows0, rows1, acc_sh, sem0, sem1):
    cid = lax.axis_index("c")
    sid = lax.axis_index("s")

    _fill_rows(rows0, CHUNK, OUT_CH, 0.0)
    for k in range(ROWS_PT // CHUNK):
        pltpu.sync_copy(rows0, acc_sh.at[pl.ds(sid * ROWS_PT + k * CHUNK, CHUNK)])
    plsc.subcore_barrier()

    base = jnp.where(cid == 0, sid * C0, NS * C0 + sid * C1)
    nstages = jnp.where(cid == 0, C0 // STG, C1 // STG)

    def stage(t, _):
        cb = base + t * STG
        pltpu.sync_copy(src_hbm.at[pl.ds(cb, STG)], src_v)
        pltpu.sync_copy(dst_hbm.at[pl.ds(cb, STG)], dst_v)
        pltpu.async_copy(hp_hbm.at[src_v.at[0]], rows0, sem0)

        def pair(i, _):
            pltpu.make_async_copy(hp_hbm.at[src_v.at[2 * i]], rows0, sem0).wait()
            pltpu.async_copy(hp_hbm.at[src_v.at[2 * i + 1]], rows1, sem1)
            pltpu.sync_copy(rows0, acc_sh.at[dst_v.at[2 * i]], add=True)
            pltpu.make_async_copy(hp_hbm.at[src_v.at[2 * i + 1]], rows1, sem1).wait()

            @pl.when(i < STG // 2 - 1)
            def _():
                pltpu.async_copy(hp_hbm.at[src_v.at[2 * i + 2]], rows0, sem0)

            pltpu.sync_copy(rows1, acc_sh.at[dst_v.at[2 * i + 1]], add=True)
            return 0

        lax.fori_loop(0, STG // 2, pair, 0)
        return 0

    lax.fori_loop(0, nstages, stage, 0)
    plsc.subcore_barrier()

    pltpu.sync_copy(
        acc_sh.at[pl.ds(sid * ROWS_PT, ROWS_PT)],
        out_hbm.at[cid, pl.ds(sid * ROWS_PT, ROWS_PT)],
    )


BLK = 1000
GRID = N_NODES // BLK


def _tc_matmul_body(x_ref, w_ref, degp_ref, hp_ref):
    deg = degp_ref[0, :, 0] + degp_ref[1, :, 0] + 1.0
    dis = lax.rsqrt(deg)
    h = jnp.dot(x_ref[...], w_ref[...], preferred_element_type=jnp.float32)
    hp_ref[...] = h * dis[:, None]


def _tc_epilogue_body(acc_ref, hp_ref, degp_ref, b_ref, o_ref):
    deg = degp_ref[0, :, 0] + degp_ref[1, :, 0] + 1.0
    dis = lax.rsqrt(deg)
    s = (acc_ref[0] + acc_ref[1] + hp_ref[...]) * dis[:, None] + b_ref[...]
    o_ref[...] = jnp.maximum(s, 0.0)


def kernel(x, edge_index, W, b):
    ei = edge_index.astype(jnp.int32)
    npad = E_PAD - N_EDGES
    src_p = jnp.concatenate([ei[0], jnp.zeros((npad,), jnp.int32)])
    dst_p = jnp.concatenate([ei[1], jnp.full((npad,), N_NODES, jnp.int32)])

    degp = _sc_degree(dst_p.reshape(NW, NCHUNK, CHUNK))
    srcA = src_p.reshape(TOT_CHUNKS, CHUNK)
    dstA = dst_p.reshape(TOT_CHUNKS, CHUNK)

    hp = pl.pallas_call(
        _tc_matmul_body,
        grid=(GRID,),
        in_specs=[
            pl.BlockSpec((BLK, D_FEAT), lambda i: (i, 0)),
            pl.BlockSpec((D_FEAT, OUT_CH), lambda i: (0, 0)),
            pl.BlockSpec((NC, BLK, CW), lambda i: (0, i, 0)),
        ],
        out_specs=pl.BlockSpec((BLK, OUT_CH), lambda i: (i, 0)),
        out_shape=jax.ShapeDtypeStruct((N_NODES, OUT_CH), jnp.float32),
    )(x, W, degp)

    acc = _sc_aggregate(srcA, dstA, hp)

    out = pl.pallas_call(
        _tc_epilogue_body,
        grid=(GRID,),
        in_specs=[
            pl.BlockSpec((NC, BLK, OUT_CH), lambda i: (0, i, 0)),
            pl.BlockSpec((BLK, OUT_CH), lambda i: (i, 0)),
            pl.BlockSpec((NC, BLK, CW), lambda i: (0, i, 0)),
            pl.BlockSpec((1, OUT_CH), lambda i: (0, 0)),
        ],
        out_specs=pl.BlockSpec((BLK, OUT_CH), lambda i: (i, 0)),
        out_shape=jax.ShapeDtypeStruct((N_NODES, OUT_CH), jnp.float32),
    )(acc, hp, degp, b.reshape(1, OUT_CH))

    return out
